# baseline (speedup 1.0000x reference)
import functools

import jax
import jax.numpy as jnp
from jax.experimental import pallas as pl
from jax.experimental.pallas import tpu as pltpu

B, L, D, F = 128, 2, 768, 32768
BF = 1024


def _body(x_ref, we_ref, be_ref, wd_ref, bd_ref, out0_ref, out1_ref):
    j = pl.program_id(0)
    f = jnp.dot(x_ref[...], we_ref[...],
                preferred_element_type=jnp.float32)
    f = jnp.maximum(f + be_ref[...], 0.0)
    p0 = jnp.dot(f, wd_ref[0],
                 preferred_element_type=jnp.float32)
    p1 = jnp.dot(f, wd_ref[1],
                 preferred_element_type=jnp.float32)

    @pl.when(j == 0)
    def _():
        out0_ref[...] = p0 + bd_ref[0][None]
        out1_ref[...] = p1 + bd_ref[1][None]

    @pl.when(j != 0)
    def _():
        out0_ref[...] += p0
        out1_ref[...] += p1


@jax.jit
def kernel(x, W_enc, b_enc, W_dec, b_dec):
    x2 = x.reshape(B, L * D)
    We = W_enc.reshape(L * D, F)
    be = b_enc.reshape(1, F)
    grid = (F // BF,)
    out0, out1 = pl.pallas_call(
        _body,
        grid=grid,
        in_specs=[
            pl.BlockSpec((B, L * D), lambda j: (0, 0)),
            pl.BlockSpec((L * D, BF), lambda j: (0, j)),
            pl.BlockSpec((1, BF), lambda j: (0, j)),
            pl.BlockSpec((L, BF, D), lambda j: (0, j, 0)),
            pl.BlockSpec((L, D), lambda j: (0, 0)),
        ],
        out_specs=[
            pl.BlockSpec((B, D), lambda j: (0, 0)),
            pl.BlockSpec((B, D), lambda j: (0, 0)),
        ],
        out_shape=[
            jax.ShapeDtypeStruct((B, D), jnp.float32),
            jax.ShapeDtypeStruct((B, D), jnp.float32),
        ],
        compiler_params=pltpu.CompilerParams(
            dimension_semantics=("arbitrary",),
        ),
    )(x2, We, be, W_dec, b_dec)
    return jnp.stack([out0, out1], axis=1)
